# baseline (speedup 1.0000x reference)
import jax
import jax.numpy as jnp
from jax.experimental import pallas as pl
from jax.experimental.pallas import tpu as pltpu

R = 20000
D = 1024
K = 117
GR = 8
BR = 1000
SL = BR // GR
LW = GR * K
NL = R // GR
NBUF = 4
NSTEP = R // BR


def _mm_kernel(x_hbm, wt_ref, b_ref, pp_ref, o_hbm, xbuf, pbuf, insem, outsem):
    i = pl.program_id(0)

    def in_copy(step, buf, j):
        return pltpu.make_async_copy(
            x_hbm.at[step, :, pl.ds(j * D, D)],
            xbuf.at[buf, j],
            insem.at[buf],
        )

    def out_copy(step, ob):
        return pltpu.make_async_copy(
            pbuf.at[ob], o_hbm.at[step], outsem.at[ob]
        )

    @pl.when(i == 0)
    def _prologue():
        for bi in range(NBUF):
            for j in range(GR):
                in_copy(bi, bi, j).start()

    buf = jax.lax.rem(i, NBUF)
    for j in range(GR):
        in_copy(i, buf, j).wait()

    packed = None
    for j in range(GR):
        acc_j = jax.lax.dot_general(
            xbuf[buf, j], wt_ref[...],
            dimension_numbers=(((1,), (0,)), ((), ())),
            preferred_element_type=jnp.float32,
        ) + b_ref[...]
        part = jax.lax.dot_general(
            acc_j, pp_ref[j],
            dimension_numbers=(((1,), (0,)), ((), ())),
            preferred_element_type=jnp.float32,
        )
        packed = part if packed is None else packed + part

    ob = jax.lax.rem(i, 2)

    @pl.when(i >= 2)
    def _wait_prev_store():
        out_copy(i - 2, ob).wait()

    pbuf[ob] = packed[:, :LW]
    out_copy(i, ob).start()

    @pl.when(i + NBUF < NSTEP)
    def _refill():
        for j in range(GR):
            in_copy(i + NBUF, buf, j).start()

    @pl.when(i == NSTEP - 1)
    def _drain():
        out_copy(i - 1, jax.lax.rem(i - 1, 2)).wait()
        out_copy(i, ob).wait()


def kernel(x, W, b):
    xv = x.reshape(NSTEP, SL, GR * D)
    wt = W.T
    bp = b.reshape(1, K)
    lanes = K * jnp.arange(GR, dtype=jnp.int32)[:, None] + jnp.arange(K, dtype=jnp.int32)[None, :]
    pp = (lanes[:, :, None] == jnp.arange(D, dtype=jnp.int32)[None, None, :]).astype(jnp.float32)
    out = pl.pallas_call(
        _mm_kernel,
        grid=(NSTEP,),
        in_specs=[
            pl.BlockSpec(memory_space=pl.ANY),
            pl.BlockSpec((D, K), lambda i: (0, 0)),
            pl.BlockSpec((1, K), lambda i: (0, 0)),
            pl.BlockSpec((GR, K, D), lambda i: (0, 0, 0)),
        ],
        out_specs=pl.BlockSpec(memory_space=pl.ANY),
        out_shape=jax.ShapeDtypeStruct((NSTEP, SL, LW), jnp.float32),
        scratch_shapes=[
            pltpu.VMEM((NBUF, GR, SL, D), jnp.float32),
            pltpu.VMEM((2, SL, LW), jnp.float32),
            pltpu.SemaphoreType.DMA((NBUF,)),
            pltpu.SemaphoreType.DMA((2,)),
        ],
        compiler_params=pltpu.CompilerParams(
            dimension_semantics=("arbitrary",),
        ),
    )(xv, wt, bp, pp)
    return out.reshape(R, K)


# speedup vs baseline: 2.5770x; 2.5770x over previous
import jax
import jax.numpy as jnp
from jax.experimental import pallas as pl
from jax.experimental.pallas import tpu as pltpu

R = 20000
D = 1024
K = 117
KP = 128
BR = 2000


def _mm_kernel(x_ref, wt_ref, b_ref, o_ref):
    acc = jax.lax.dot_general(
        x_ref[...], wt_ref[...],
        dimension_numbers=(((1,), (0,)), ((), ())),
        preferred_element_type=jnp.float32,
    )
    o_ref[...] = acc + b_ref[...]


def kernel(x, W, b):
    wt = jnp.concatenate([W, jnp.zeros((KP - K, D), jnp.float32)], axis=0).T
    bp = jnp.concatenate([b, jnp.zeros((KP - K,), jnp.float32)]).reshape(1, KP)
    padded = pl.pallas_call(
        _mm_kernel,
        grid=(R // BR,),
        in_specs=[
            pl.BlockSpec((BR, D), lambda i: (i, 0)),
            pl.BlockSpec((D, KP), lambda i: (0, 0)),
            pl.BlockSpec((1, KP), lambda i: (0, 0)),
        ],
        out_specs=pl.BlockSpec((BR, KP), lambda i: (i, 0)),
        out_shape=jax.ShapeDtypeStruct((R, KP), jnp.float32),
        compiler_params=pltpu.CompilerParams(
            dimension_semantics=("arbitrary",),
        ),
    )(x, wt, bp)
    return padded[:, :K] * jnp.float32(1.0)


# speedup vs baseline: 3.5187x; 1.3654x over previous
import jax
import jax.numpy as jnp
from jax.experimental import pallas as pl
from jax.experimental.pallas import tpu as pltpu

R = 20000
D = 1024
K = 117
BR = 4000


def _mm_kernel(x_ref, wt_ref, b_ref, o_ref):
    acc = jax.lax.dot_general(
        x_ref[...], wt_ref[...],
        dimension_numbers=(((1,), (0,)), ((), ())),
        preferred_element_type=jnp.float32,
    )
    o_ref[...] = acc + b_ref[...]


def kernel(x, W, b):
    wt = W.T
    bp = b.reshape(1, K)
    return pl.pallas_call(
        _mm_kernel,
        grid=(R // BR,),
        in_specs=[
            pl.BlockSpec((BR, D), lambda i: (i, 0)),
            pl.BlockSpec((D, K), lambda i: (0, 0)),
            pl.BlockSpec((1, K), lambda i: (0, 0)),
        ],
        out_specs=pl.BlockSpec((BR, K), lambda i: (i, 0)),
        out_shape=jax.ShapeDtypeStruct((R, K), jnp.float32),
        compiler_params=pltpu.CompilerParams(
            dimension_semantics=("arbitrary",),
        ),
    )(x, wt, bp)


# speedup vs baseline: 3.6182x; 1.0283x over previous
import jax
import jax.numpy as jnp
from jax.experimental import pallas as pl
from jax.experimental.pallas import tpu as pltpu

R = 20000
D = 1024
K = 117
BR = 2000


def _mm_kernel(x_ref, wt_ref, b_ref, o_ref):
    acc = jax.lax.dot_general(
        x_ref[...], wt_ref[...],
        dimension_numbers=(((1,), (0,)), ((), ())),
        preferred_element_type=jnp.float32,
    )
    o_ref[...] = acc + b_ref[...]


def kernel(x, W, b):
    wt = W.T
    bp = b.reshape(1, K)
    return pl.pallas_call(
        _mm_kernel,
        grid=(R // BR,),
        in_specs=[
            pl.BlockSpec((BR, D), lambda i: (i, 0)),
            pl.BlockSpec((D, K), lambda i: (0, 0)),
            pl.BlockSpec((1, K), lambda i: (0, 0)),
        ],
        out_specs=pl.BlockSpec((BR, K), lambda i: (i, 0)),
        out_shape=jax.ShapeDtypeStruct((R, K), jnp.float32),
        compiler_params=pltpu.CompilerParams(
            dimension_semantics=("arbitrary",),
        ),
    )(x, wt, bp)
